# baseline (speedup 1.0000x reference)
import functools

import jax
import jax.numpy as jnp
from jax import lax
from jax.experimental import pallas as pl
from jax.experimental.pallas import tpu as pltpu
from jax.experimental.pallas import tpu_sc as plsc

LL = 3
NN = 50000
EE = 800000
BB = 16384
HH = 64

_K = 1600
_NBLK = EE // _K
_KA = 400
_NBLKA = EE // _KA
_ZCH = 2000
_ZR = 200

_mesh = plsc.VectorSubcoreMesh(core_axis_name="c", subcore_axis_name="s")


def _take16(x, idx):
    return lax.gather(
        x, idx[:, None],
        dimension_numbers=lax.GatherDimensionNumbers(
            offset_dims=(), collapsed_slice_dims=(0,), start_index_map=(0,)),
        slice_sizes=(1,), mode=lax.GatherScatterMode.PROMISE_IN_BOUNDS)


def _prefix16(x):
    iota = lax.iota(jnp.int32, 16)
    for d in (1, 2, 4, 8):
        sh = _take16(x, jnp.maximum(iota - d, 0))
        gate = 1 - (lax.shift_right_arithmetic(iota - d, 31) & 1)
        x = x + sh * gate
    return x
_sc_params = pltpu.CompilerParams(use_tc_tiling_on_sc=False)


def _fill(ref, n, val, dtype):
    def body(i, _):
        ref[pl.ds(i * 16, 16)] = jnp.full((16,), val, dtype)
        return 0
    lax.fori_loop(0, n // 16, body, 0)


def _sc_deg_body(dst_hbm, out_hbm, idx_v, ones_v, zb_v, stage_v, a0, a1, a2):
    accs = (a0, a1, a2)
    c = lax.axis_index("c")
    s = lax.axis_index("s")
    w = c * 16 + s
    _fill(ones_v, _K, 1.0, jnp.float32)
    _fill(zb_v, _ZCH, 0.0, jnp.float32)
    for l in range(LL):
        def zbody(k, _, acc=accs[l]):
            ch = s + 16 * k
            @pl.when(ch < NN // _ZCH)
            def _():
                pltpu.sync_copy(zb_v, acc.at[pl.ds(ch * _ZCH, _ZCH)])
            return 0
        lax.fori_loop(0, 2, zbody, 0)
    plsc.subcore_barrier()
    for l in range(LL):
        def ebody(k, _, l=l, acc=accs[l]):
            b = w + 32 * k
            @pl.when(b < _NBLK)
            def _():
                pltpu.sync_copy(dst_hbm.at[pl.ds(l * EE + b * _K, _K)], idx_v)
                pltpu.sync_copy(ones_v, acc.at[idx_v], add=True)
            return 0
        lax.fori_loop(0, 16, ebody, 0)
    plsc.subcore_barrier()
    for l in range(LL):
        def obody(k, _, l=l, acc=accs[l]):
            ch = s + 16 * k
            @pl.when(ch < NN // _ZCH)
            def _():
                pltpu.sync_copy(acc.at[pl.ds(ch * _ZCH, _ZCH)], stage_v)
                pltpu.sync_copy(
                    stage_v,
                    out_hbm.at[pl.ds((c * LL + l) * NN + ch * _ZCH, _ZCH)])
            return 0
        lax.fori_loop(0, 2, obody, 0)


_sc_deg = functools.partial(
    pl.kernel, _sc_deg_body,
    compiler_params=_sc_params,
    out_type=jax.ShapeDtypeStruct((2 * LL * NN,), jnp.float32),
    mesh=_mesh,
    scratch_types=[
        pltpu.VMEM((_K,), jnp.int32),
        pltpu.VMEM((_K,), jnp.float32),
        pltpu.VMEM((_ZCH,), jnp.float32),
        pltpu.VMEM((_ZCH,), jnp.float32),
        pltpu.VMEM_SHARED((NN,), jnp.float32),
        pltpu.VMEM_SHARED((NN,), jnp.float32),
        pltpu.VMEM_SHARED((NN,), jnp.float32),
    ],
)()


_KT = 3200
_NBLKT = EE // _KT

def _sc_txsum_body(src_hbm, dst_hbm, xd_hbm, out_hbm,
                   si_v, di_v, val_v, zb_v, stage_v, xd_sh, a0, a1, a2):
    accs = (a0, a1, a2)
    c = lax.axis_index("c")
    s = lax.axis_index("s")
    w = c * 16 + s
    _fill(zb_v, _ZCH, 0.0, jnp.float32)
    for l in range(LL):
        def zbody(k, _, acc=accs[l]):
            ch = s + 16 * k
            @pl.when(ch < NN // _ZCH)
            def _():
                pltpu.sync_copy(zb_v, acc.at[pl.ds(ch * _ZCH, _ZCH)])
            return 0
        lax.fori_loop(0, 2, zbody, 0)
    for l in range(LL):
        def sbody(k, _, l=l):
            ch = s + 16 * k
            @pl.when(ch < NN // _ZCH)
            def _():
                pltpu.sync_copy(
                    xd_hbm.at[pl.ds(l * NN + ch * _ZCH, _ZCH)], stage_v)
                pltpu.sync_copy(stage_v, xd_sh.at[pl.ds(ch * _ZCH, _ZCH)])
            return 0
        lax.fori_loop(0, 2, sbody, 0)
        plsc.subcore_barrier()
        def ebody(k, _, l=l, acc=accs[l]):
            b = w + 32 * k
            @pl.when(b < _NBLKT)
            def _():
                pltpu.sync_copy(src_hbm.at[pl.ds(l * EE + b * _KT, _KT)], si_v)
                pltpu.sync_copy(dst_hbm.at[pl.ds(l * EE + b * _KT, _KT)], di_v)
                pltpu.sync_copy(xd_sh.at[si_v], val_v)
                pltpu.sync_copy(val_v, acc.at[di_v], add=True)
            return 0
        lax.fori_loop(0, 8, ebody, 0)
        plsc.subcore_barrier()
    for l in range(LL):
        def obody(k, _, l=l, acc=accs[l]):
            ch = s + 16 * k
            @pl.when(ch < NN // _ZCH)
            def _():
                pltpu.sync_copy(acc.at[pl.ds(ch * _ZCH, _ZCH)], stage_v)
                pltpu.sync_copy(
                    stage_v,
                    out_hbm.at[pl.ds((c * LL + l) * NN + ch * _ZCH, _ZCH)])
            return 0
        lax.fori_loop(0, 2, obody, 0)


_sc_txsum = functools.partial(
    pl.kernel, _sc_txsum_body,
    compiler_params=_sc_params,
    out_type=jax.ShapeDtypeStruct((2 * LL * NN,), jnp.float32),
    mesh=_mesh,
    scratch_types=[
        pltpu.VMEM((_KT,), jnp.int32),
        pltpu.VMEM((_KT,), jnp.int32),
        pltpu.VMEM((_KT,), jnp.float32),
        pltpu.VMEM((_ZCH,), jnp.float32),
        pltpu.VMEM((_ZCH,), jnp.float32),
        pltpu.VMEM_SHARED((NN,), jnp.float32),
        pltpu.VMEM_SHARED((NN,), jnp.float32),
        pltpu.VMEM_SHARED((NN,), jnp.float32),
        pltpu.VMEM_SHARED((NN,), jnp.float32),
    ],
)()


_KA = 320
_NBLKA = EE // _KA
_ZRA = 200

def _sc_agg_body(src_hbm, dst_hbm, g_hbm, out_hbm,
                 si0, di0, fi0, rows0, si1, di1, fi1, rows1,
                 stage_v, acc, semg, sems0, sems1):
    c = lax.axis_index("c")
    s = lax.axis_index("s")
    bufs = ((si0, di0, fi0, rows0, sems0), (si1, di1, fi1, rows1, sems1))

    def zfill():
        def zr(i, _):
            z16 = jnp.zeros((16,), jnp.float32)
            stage_v[i, pl.ds(0, 16)] = z16
            stage_v[i, pl.ds(16, 16)] = z16
            return 0
        lax.fori_loop(0, _ZRA, zr, 0)

    def zacc():
        def zbody(k, _):
            ch = s + 16 * k
            @pl.when(ch < NN // _ZRA)
            def _():
                pltpu.sync_copy(stage_v, acc.at[pl.ds(ch * _ZRA, _ZRA)])
            return 0
        lax.fori_loop(0, (NN // _ZRA + 15) // 16, zbody, 0)

    zfill()
    zacc()
    plsc.subcore_barrier()
    for l in range(LL):
        base = (2 * l) * NN + c * NN
        def ebody(k2, _, l=l, base=base):
            for j in (0, 1):
                si_v, di_v, fi_v, rows_v, sems = bufs[j]
                bb = s + 16 * (2 * k2 + j)
                @pl.when(bb < _NBLKA)
                def _(bb=bb, si_v=si_v, di_v=di_v, fi_v=fi_v, rows_v=rows_v,
                      sems=sems, k2=k2):
                    pltpu.sync_copy(
                        src_hbm.at[pl.ds(l * EE + bb * _KA, _KA)], si_v)
                    pltpu.sync_copy(
                        dst_hbm.at[pl.ds(l * EE + bb * _KA, _KA)], di_v)
                    def fbody(i, _):
                        fi_v[pl.ds(i * 16, 16)] = (
                            si_v[pl.ds(i * 16, 16)] + base)
                        return 0
                    lax.fori_loop(0, _KA // 16, fbody, 0)
                    pltpu.async_copy(g_hbm.at[fi_v], rows_v, semg).wait()
                    pltpu.sync_copy(rows_v, acc.at[di_v], add=True)
            return 0
        lax.fori_loop(0, (_NBLKA + 31) // 32, ebody, 0)
        plsc.subcore_barrier()
        def obody(k, _, l=l):
            ch = s + 16 * k
            @pl.when(ch < NN // _ZRA)
            def _():
                pltpu.sync_copy(acc.at[pl.ds(ch * _ZRA, _ZRA)], stage_v)
                pltpu.sync_copy(
                    stage_v,
                    out_hbm.at[pl.ds((2 * l + c) * NN + ch * _ZRA, _ZRA)])
            return 0
        lax.fori_loop(0, (NN // _ZRA + 15) // 16, obody, 0)
        if l != LL - 1:
            zfill()
            zacc()
        plsc.subcore_barrier()


_sc_agg = functools.partial(
    pl.kernel, _sc_agg_body,
    compiler_params=_sc_params,
    out_type=jax.ShapeDtypeStruct((2 * LL * NN, 32), jnp.float32),
    mesh=_mesh,
    scratch_types=[
        pltpu.VMEM((_KA,), jnp.int32),
        pltpu.VMEM((_KA,), jnp.int32),
        pltpu.VMEM((_KA,), jnp.int32),
        pltpu.VMEM((_KA, 32), jnp.float32),
        pltpu.VMEM((_KA,), jnp.int32),
        pltpu.VMEM((_KA,), jnp.int32),
        pltpu.VMEM((_KA,), jnp.int32),
        pltpu.VMEM((_KA, 32), jnp.float32),
        pltpu.VMEM((_ZRA, 32), jnp.float32),
        pltpu.VMEM_SHARED((NN, 32), jnp.float32),
        pltpu.SemaphoreType.DMA,
        pltpu.SemaphoreType.DMA,
        pltpu.SemaphoreType.DMA,
    ],
)()


_QC = 512

def _sc_headg_body(ql_hbm, qn_hbm, u_hbm, s_hbm, d_hbm,
                   u0_out, u1_out, sq_out, dq_out,
                   l_v, n_v, f0_v, f1_v, fs_v, r0_v, r1_v, sv_v, dv_v, sem):
    c = lax.axis_index("c")
    s = lax.axis_index("s")
    w = c * 16 + s

    def chunk(ch, _):
        base = w * (2 * _QC) + ch * _QC
        pltpu.sync_copy(ql_hbm.at[pl.ds(base, _QC)], l_v)
        pltpu.sync_copy(qn_hbm.at[pl.ds(base, _QC)], n_v)
        def fbody(i, _):
            l16 = l_v[pl.ds(i * 16, 16)]
            n16 = n_v[pl.ds(i * 16, 16)]
            fs_v[pl.ds(i * 16, 16)] = l16 * NN + n16
            f0 = l16 * (2 * NN) + n16
            f0_v[pl.ds(i * 16, 16)] = f0
            f1_v[pl.ds(i * 16, 16)] = f0 + NN
            return 0
        lax.fori_loop(0, _QC // 16, fbody, 0)
        pltpu.async_copy(u_hbm.at[f0_v], r0_v, sem).wait()
        pltpu.async_copy(u_hbm.at[f1_v], r1_v, sem).wait()
        pltpu.async_copy(s_hbm.at[fs_v], sv_v, sem).wait()
        pltpu.async_copy(d_hbm.at[fs_v], dv_v, sem).wait()
        pltpu.sync_copy(r0_v, u0_out.at[pl.ds(base, _QC)])
        pltpu.sync_copy(r1_v, u1_out.at[pl.ds(base, _QC)])
        pltpu.sync_copy(sv_v, sq_out.at[pl.ds(base, _QC)])
        pltpu.sync_copy(dv_v, dq_out.at[pl.ds(base, _QC)])
        return 0

    lax.fori_loop(0, 2, chunk, 0)


_sc_headg = functools.partial(
    pl.kernel, _sc_headg_body,
    compiler_params=_sc_params,
    out_type=[
        jax.ShapeDtypeStruct((2 * BB, 32), jnp.float32),
        jax.ShapeDtypeStruct((2 * BB, 32), jnp.float32),
        jax.ShapeDtypeStruct((2 * BB,), jnp.float32),
        jax.ShapeDtypeStruct((2 * BB,), jnp.float32),
    ],
    mesh=_mesh,
    scratch_types=[
        pltpu.VMEM((_QC,), jnp.int32),
        pltpu.VMEM((_QC,), jnp.int32),
        pltpu.VMEM((_QC,), jnp.int32),
        pltpu.VMEM((_QC,), jnp.int32),
        pltpu.VMEM((_QC,), jnp.int32),
        pltpu.VMEM((_QC, 32), jnp.float32),
        pltpu.VMEM((_QC, 32), jnp.float32),
        pltpu.VMEM((_QC,), jnp.float32),
        pltpu.VMEM((_QC,), jnp.float32),
        pltpu.SemaphoreType.DMA,
    ],
)()


def _tc_stats_body(t_ref, x_ref, dinv_ref, s_ref, st_ref):
    dinv = dinv_ref[...]
    sarr = dinv * t_ref[...] + dinv * dinv * x_ref[...]
    s_ref[...] = sarr
    st_ref[...] = jnp.stack(
        [jnp.sum(sarr, axis=1), jnp.sum(sarr * sarr, axis=1)], axis=1)


def _tc_stats(t, x, dinv):
    return pl.pallas_call(
        _tc_stats_body,
        out_shape=[
            jax.ShapeDtypeStruct((LL, NN), jnp.float32),
            jax.ShapeDtypeStruct((LL, 2), jnp.float32),
        ],
    )(t, x, dinv)


_GB = 2000

def _tc_g_body(s_ref, dinv_ref, a_ref, c_ref, g_ref):
    q = 2 * pl.program_id(0) + pl.program_id(1)
    ab = a_ref[q, :]
    cb = c_ref[q, :]
    sb = s_ref[...]
    db = dinv_ref[...]
    g_ref[...] = db * jnp.maximum(sb * ab[None, :] + cb[None, :], 0.0)


def _tc_g(s2d, dinv2d, a, c):
    return pl.pallas_call(
        _tc_g_body,
        grid=(LL, 2, NN // _GB),
        in_specs=[
            pl.BlockSpec((_GB, 1), lambda l, h, nb: (l * (NN // _GB) + nb, 0)),
            pl.BlockSpec((_GB, 1), lambda l, h, nb: (l * (NN // _GB) + nb, 0)),
            pl.BlockSpec((2 * LL, 32), lambda l, h, nb: (0, 0)),
            pl.BlockSpec((2 * LL, 32), lambda l, h, nb: (0, 0)),
        ],
        out_specs=pl.BlockSpec(
            (_GB, 32), lambda l, h, nb: ((2 * l + h) * (NN // _GB) + nb, 0)),
        out_shape=jax.ShapeDtypeStruct((2 * LL * NN, 32), jnp.float32),
    )(s2d, dinv2d, a, c)


_HB = 1024

def _tc_head_body(u0L, u0R, u1L, u1R, sqL, sqR, dqL, dqR,
                  lay_ref, ln_ref, rn_ref,
                  a_ref, c_ref, W2_ref, b2_ref,
                  gW1_ref, gb1_ref, gW2_ref, gb2_ref,
                  wsm_ref, wss_ref, wsb_ref,
                  dW_ref, db_ref, pW_ref, pb_ref,
                  p_ref, d_ref):
    lay = lay_ref[...]
    oh = (lay == jnp.arange(LL, dtype=jnp.int32)[None, :]).astype(jnp.float32)
    aR = oh @ a_ref[...]
    cR = oh @ c_ref[...]
    b2R = oh @ b2_ref[...]

    def side(u0, u1, sq, dq):
        u = jnp.concatenate([u0[...], u1[...]], axis=1)
        sv = sq[...]
        dv = dq[...]
        h = jnp.maximum(aR * sv + cR, 0.0)
        pre = dv * u + dv * dv * h
        emb = b2R
        for l in range(LL):
            emb = emb + oh[:, l:l + 1] * (pre @ W2_ref[l])
        return emb

    embL = side(u0L, u1L, sqL, dqL)
    embR = side(u0R, u1R, sqR, dqR)
    specific = jnp.concatenate([embL, embR], axis=1)
    common = (jnp.maximum(specific @ gW1_ref[...] + gb1_ref[...], 0.0)
              @ gW2_ref[...] + gb2_ref[...])
    layf = lay.astype(jnp.float32)
    lnf = ln_ref[...].astype(jnp.float32)
    rnf = rn_ref[...].astype(jnp.float32)
    wsm = wsm_ref[...]
    z = (layf * wsm[0:1, :] + lnf * wsm[1:2, :] + rnf * wsm[2:3, :]
         + specific @ wss_ref[...] + wsb_ref[...])
    z = z - jnp.max(z, axis=1, keepdims=True)
    ez = jnp.exp(z)
    wsf = ez / jnp.sum(ez, axis=1, keepdims=True)
    p_in = specific * wsf[:, 0:1] + common * wsf[:, 1:2]
    p_ref[...] = p_in @ pW_ref[...] + pb_ref[...]
    d_ref[...] = common @ dW_ref[...] + db_ref[...]


def _tc_head(u0, u1, sq, dq, lay2, ln2, rn2, a, c, W2, b2,
             gW1, gb1, gW2, gb2, wsm, wss, wsb, dW, db, pW, pb):
    nb = BB // _HB
    left = lambda i: (i, 0)
    right = lambda i: (i + nb, 0)
    full = lambda i: tuple([0] * 2)
    full3 = lambda i: (0, 0, 0)
    return pl.pallas_call(
        _tc_head_body,
        grid=(nb,),
        in_specs=[
            pl.BlockSpec((_HB, 32), left), pl.BlockSpec((_HB, 32), right),
            pl.BlockSpec((_HB, 32), left), pl.BlockSpec((_HB, 32), right),
            pl.BlockSpec((_HB, 1), left), pl.BlockSpec((_HB, 1), right),
            pl.BlockSpec((_HB, 1), left), pl.BlockSpec((_HB, 1), right),
            pl.BlockSpec((_HB, 1), left),
            pl.BlockSpec((_HB, 1), left),
            pl.BlockSpec((_HB, 1), left),
            pl.BlockSpec((LL, HH), full), pl.BlockSpec((LL, HH), full),
            pl.BlockSpec((LL, HH, HH), full3), pl.BlockSpec((LL, HH), full),
            pl.BlockSpec((2 * HH, 16), full), pl.BlockSpec((1, 16), full),
            pl.BlockSpec((16, 2 * HH), full), pl.BlockSpec((1, 2 * HH), full),
            pl.BlockSpec((3, 2), full), pl.BlockSpec((2 * HH, 2), full),
            pl.BlockSpec((1, 2), full),
            pl.BlockSpec((2 * HH, LL), full), pl.BlockSpec((1, LL), full),
            pl.BlockSpec((2 * HH, 2), full), pl.BlockSpec((1, 2), full),
        ],
        out_specs=[
            pl.BlockSpec((_HB, 2), left),
            pl.BlockSpec((_HB, LL), left),
        ],
        out_shape=[
            jax.ShapeDtypeStruct((BB, 2), jnp.float32),
            jax.ShapeDtypeStruct((BB, LL), jnp.float32),
        ],
    )(u0, u0, u1, u1, sq, sq, dq, dq, lay2[:BB], ln2, rn2, a, c, W2, b2,
      gW1, gb1, gW2, gb2, wsm, wss, wsb, dW, db, pW, pb)


def kernel(xs, edge_index, leftnode, rightnode, layer,
           W1, b1, gamma, beta, W2, b2,
           gW1, gb1, gW2, gb2, wsW, wsb, dW, db, pW, pb):
    x = xs[..., 0]
    src = edge_index[:, 0, :].reshape(LL * EE)
    dst = edge_index[:, 1, :].reshape(LL * EE)

    degp = _sc_deg(dst).reshape(2, LL, NN)
    deg = degp[0] + degp[1] + 1.0
    dinv = lax.rsqrt(deg)
    xd = (x * dinv).reshape(LL * NN)

    tp = _sc_txsum(src, dst, xd).reshape(2, LL, NN)
    t = tp[0] + tp[1]

    s_arr, st = _tc_stats(t, x, dinv)
    mean = st[:, 0] / NN
    var = st[:, 1] / NN - mean * mean
    w1r = W1[:, 0, :]
    a = gamma * w1r * lax.rsqrt(var[:, None] * w1r * w1r + 1e-5)
    c = beta - a * mean[:, None]

    g = _tc_g(s_arr.reshape(LL * NN, 1), dinv.reshape(LL * NN, 1),
              a.reshape(2 * LL, 32), c.reshape(2 * LL, 32))
    u = _sc_agg(src, dst, g)

    ql = jnp.concatenate([layer, layer])
    qn = jnp.concatenate([leftnode, rightnode])
    u0, u1, sq, dq = _sc_headg(
        ql, qn, u, s_arr.reshape(LL * NN), dinv.reshape(LL * NN))

    p_out, d_out = _tc_head(
        u0, u1, sq.reshape(2 * BB, 1), dq.reshape(2 * BB, 1),
        ql.reshape(2 * BB, 1), leftnode.reshape(BB, 1),
        rightnode.reshape(BB, 1),
        a, c, W2, b2,
        gW1, gb1.reshape(1, 16), gW2, gb2.reshape(1, 2 * HH),
        wsW[:3], wsW[3:], wsb.reshape(1, 2),
        dW, db.reshape(1, LL), pW, pb.reshape(1, 2))
    return (p_out, d_out)


# speedup vs baseline: 1.2804x; 1.2804x over previous
import functools

import jax
import jax.numpy as jnp
from jax import lax
from jax.experimental import pallas as pl
from jax.experimental.pallas import tpu as pltpu
from jax.experimental.pallas import tpu_sc as plsc

LL = 3
NN = 50000
EE = 800000
BB = 16384
HH = 64

_K = 1600
_NBLK = EE // _K
_KA = 400
_NBLKA = EE // _KA
_ZCH = 2000
_ZR = 200

_mesh = plsc.VectorSubcoreMesh(core_axis_name="c", subcore_axis_name="s")


def _take16(x, idx):
    return lax.gather(
        x, idx[:, None],
        dimension_numbers=lax.GatherDimensionNumbers(
            offset_dims=(), collapsed_slice_dims=(0,), start_index_map=(0,)),
        slice_sizes=(1,), mode=lax.GatherScatterMode.PROMISE_IN_BOUNDS)


def _prefix16(x):
    iota = lax.iota(jnp.int32, 16)
    for d in (1, 2, 4, 8):
        sh = _take16(x, jnp.maximum(iota - d, 0))
        gate = 1 - (lax.shift_right_arithmetic(iota - d, 31) & 1)
        x = x + sh * gate
    return x
_sc_params = pltpu.CompilerParams(use_tc_tiling_on_sc=False)


def _fill(ref, n, val, dtype):
    def body(i, _):
        ref[pl.ds(i * 16, 16)] = jnp.full((16,), val, dtype)
        return 0
    lax.fori_loop(0, n // 16, body, 0)


def _sc_deg_body(dst_hbm, out_hbm, idx_v, ones_v, zb_v, stage_v, a0, a1, a2):
    accs = (a0, a1, a2)
    c = lax.axis_index("c")
    s = lax.axis_index("s")
    w = c * 16 + s
    _fill(ones_v, _K, 1.0, jnp.float32)
    _fill(zb_v, _ZCH, 0.0, jnp.float32)
    for l in range(LL):
        def zbody(k, _, acc=accs[l]):
            ch = s + 16 * k
            @pl.when(ch < NN // _ZCH)
            def _():
                pltpu.sync_copy(zb_v, acc.at[pl.ds(ch * _ZCH, _ZCH)])
            return 0
        lax.fori_loop(0, 2, zbody, 0)
    plsc.subcore_barrier()
    for l in range(LL):
        def ebody(k, _, l=l, acc=accs[l]):
            b = w + 32 * k
            @pl.when(b < _NBLK)
            def _():
                pltpu.sync_copy(dst_hbm.at[pl.ds(l * EE + b * _K, _K)], idx_v)
                pltpu.sync_copy(ones_v, acc.at[idx_v], add=True)
            return 0
        lax.fori_loop(0, 16, ebody, 0)
    plsc.subcore_barrier()
    for l in range(LL):
        def obody(k, _, l=l, acc=accs[l]):
            ch = s + 16 * k
            @pl.when(ch < NN // _ZCH)
            def _():
                pltpu.sync_copy(acc.at[pl.ds(ch * _ZCH, _ZCH)], stage_v)
                pltpu.sync_copy(
                    stage_v,
                    out_hbm.at[pl.ds((c * LL + l) * NN + ch * _ZCH, _ZCH)])
            return 0
        lax.fori_loop(0, 2, obody, 0)


_sc_deg = functools.partial(
    pl.kernel, _sc_deg_body,
    compiler_params=_sc_params,
    out_type=jax.ShapeDtypeStruct((2 * LL * NN,), jnp.float32),
    mesh=_mesh,
    scratch_types=[
        pltpu.VMEM((_K,), jnp.int32),
        pltpu.VMEM((_K,), jnp.float32),
        pltpu.VMEM((_ZCH,), jnp.float32),
        pltpu.VMEM((_ZCH,), jnp.float32),
        pltpu.VMEM_SHARED((NN,), jnp.float32),
        pltpu.VMEM_SHARED((NN,), jnp.float32),
        pltpu.VMEM_SHARED((NN,), jnp.float32),
    ],
)()


_KT = 3200
_NBLKT = EE // _KT

def _sc_txsum_body(src_hbm, dst_hbm, xd_hbm, out_hbm,
                   si_v, di_v, val_v, zb_v, stage_v, xd_sh, a0, a1, a2):
    accs = (a0, a1, a2)
    c = lax.axis_index("c")
    s = lax.axis_index("s")
    w = c * 16 + s
    _fill(zb_v, _ZCH, 0.0, jnp.float32)
    for l in range(LL):
        def zbody(k, _, acc=accs[l]):
            ch = s + 16 * k
            @pl.when(ch < NN // _ZCH)
            def _():
                pltpu.sync_copy(zb_v, acc.at[pl.ds(ch * _ZCH, _ZCH)])
            return 0
        lax.fori_loop(0, 2, zbody, 0)
    for l in range(LL):
        def sbody(k, _, l=l):
            ch = s + 16 * k
            @pl.when(ch < NN // _ZCH)
            def _():
                pltpu.sync_copy(
                    xd_hbm.at[pl.ds(l * NN + ch * _ZCH, _ZCH)], stage_v)
                pltpu.sync_copy(stage_v, xd_sh.at[pl.ds(ch * _ZCH, _ZCH)])
            return 0
        lax.fori_loop(0, 2, sbody, 0)
        plsc.subcore_barrier()
        def ebody(k, _, l=l, acc=accs[l]):
            b = w + 32 * k
            @pl.when(b < _NBLKT)
            def _():
                pltpu.sync_copy(src_hbm.at[pl.ds(l * EE + b * _KT, _KT)], si_v)
                pltpu.sync_copy(dst_hbm.at[pl.ds(l * EE + b * _KT, _KT)], di_v)
                pltpu.sync_copy(xd_sh.at[si_v], val_v)
                pltpu.sync_copy(val_v, acc.at[di_v], add=True)
            return 0
        lax.fori_loop(0, 8, ebody, 0)
        plsc.subcore_barrier()
    for l in range(LL):
        def obody(k, _, l=l, acc=accs[l]):
            ch = s + 16 * k
            @pl.when(ch < NN // _ZCH)
            def _():
                pltpu.sync_copy(acc.at[pl.ds(ch * _ZCH, _ZCH)], stage_v)
                pltpu.sync_copy(
                    stage_v,
                    out_hbm.at[pl.ds((c * LL + l) * NN + ch * _ZCH, _ZCH)])
            return 0
        lax.fori_loop(0, 2, obody, 0)


_sc_txsum = functools.partial(
    pl.kernel, _sc_txsum_body,
    compiler_params=_sc_params,
    out_type=jax.ShapeDtypeStruct((2 * LL * NN,), jnp.float32),
    mesh=_mesh,
    scratch_types=[
        pltpu.VMEM((_KT,), jnp.int32),
        pltpu.VMEM((_KT,), jnp.int32),
        pltpu.VMEM((_KT,), jnp.float32),
        pltpu.VMEM((_ZCH,), jnp.float32),
        pltpu.VMEM((_ZCH,), jnp.float32),
        pltpu.VMEM_SHARED((NN,), jnp.float32),
        pltpu.VMEM_SHARED((NN,), jnp.float32),
        pltpu.VMEM_SHARED((NN,), jnp.float32),
        pltpu.VMEM_SHARED((NN,), jnp.float32),
    ],
)()


_KA = 320
_NBLKA = EE // _KA
_ZRA = 200

def _sc_agg_body(src_hbm, dst_hbm, g_hbm, out_hbm,
                 si0, di0, fi0, rows0, si1, di1, fi1, rows1,
                 stage_v, acc, semg, sems0, sems1):
    c = lax.axis_index("c")
    s = lax.axis_index("s")
    bufs = ((si0, di0, fi0, rows0, sems0), (si1, di1, fi1, rows1, sems1))

    def zfill():
        def zr(i, _):
            z16 = jnp.zeros((16,), jnp.float32)
            stage_v[i, pl.ds(0, 16)] = z16
            stage_v[i, pl.ds(16, 16)] = z16
            return 0
        lax.fori_loop(0, _ZRA, zr, 0)

    def zacc():
        def zbody(k, _):
            ch = s + 16 * k
            @pl.when(ch < NN // _ZRA)
            def _():
                pltpu.sync_copy(stage_v, acc.at[pl.ds(ch * _ZRA, _ZRA)])
            return 0
        lax.fori_loop(0, (NN // _ZRA + 15) // 16, zbody, 0)

    zfill()
    zacc()
    plsc.subcore_barrier()

    def load_and_fire(bb, l, base, si_v, di_v, fi_v, rows_v, sems):
        pltpu.sync_copy(src_hbm.at[pl.ds(l * EE + bb * _KA, _KA)], si_v)
        pltpu.sync_copy(dst_hbm.at[pl.ds(l * EE + bb * _KA, _KA)], di_v)
        def fbody(i, _):
            fi_v[pl.ds(i * 16, 16)] = si_v[pl.ds(i * 16, 16)] + base
            return 0
        lax.fori_loop(0, _KA // 16, fbody, 0)
        pltpu.async_copy(g_hbm.at[fi_v], rows_v, sems)

    for l in range(LL):
        base = (2 * l) * NN + c * NN
        for j in (0, 1):
            si_v, di_v, fi_v, rows_v, sems = bufs[j]
            bb = s + 16 * j
            @pl.when(bb < _NBLKA)
            def _(bb=bb, si_v=si_v, di_v=di_v, fi_v=fi_v, rows_v=rows_v,
                  sems=sems):
                load_and_fire(bb, l, base, si_v, di_v, fi_v, rows_v, sems)
        def ebody(k2, _, l=l, base=base):
            for j in (0, 1):
                si_v, di_v, fi_v, rows_v, sems = bufs[j]
                bb = s + 16 * (2 * k2 + j)
                @pl.when(bb < _NBLKA)
                def _(bb=bb, si_v=si_v, di_v=di_v, fi_v=fi_v, rows_v=rows_v,
                      sems=sems):
                    pltpu.make_async_copy(g_hbm.at[fi_v], rows_v, sems).wait()
                    pltpu.sync_copy(rows_v, acc.at[di_v], add=True)
                    @pl.when(bb + 32 < _NBLKA)
                    def _():
                        load_and_fire(bb + 32, l, base,
                                      si_v, di_v, fi_v, rows_v, sems)
            return 0
        lax.fori_loop(0, (_NBLKA + 31) // 32, ebody, 0)
        plsc.subcore_barrier()
        def obody(k, _, l=l):
            ch = s + 16 * k
            @pl.when(ch < NN // _ZRA)
            def _():
                pltpu.sync_copy(acc.at[pl.ds(ch * _ZRA, _ZRA)], stage_v)
                pltpu.sync_copy(
                    stage_v,
                    out_hbm.at[pl.ds((2 * l + c) * NN + ch * _ZRA, _ZRA)])
            return 0
        lax.fori_loop(0, (NN // _ZRA + 15) // 16, obody, 0)
        if l != LL - 1:
            zfill()
            zacc()
        plsc.subcore_barrier()


_sc_agg = functools.partial(
    pl.kernel, _sc_agg_body,
    compiler_params=_sc_params,
    out_type=jax.ShapeDtypeStruct((2 * LL * NN, 32), jnp.float32),
    mesh=_mesh,
    scratch_types=[
        pltpu.VMEM((_KA,), jnp.int32),
        pltpu.VMEM((_KA,), jnp.int32),
        pltpu.VMEM((_KA,), jnp.int32),
        pltpu.VMEM((_KA, 32), jnp.float32),
        pltpu.VMEM((_KA,), jnp.int32),
        pltpu.VMEM((_KA,), jnp.int32),
        pltpu.VMEM((_KA,), jnp.int32),
        pltpu.VMEM((_KA, 32), jnp.float32),
        pltpu.VMEM((_ZRA, 32), jnp.float32),
        pltpu.VMEM_SHARED((NN, 32), jnp.float32),
        pltpu.SemaphoreType.DMA,
        pltpu.SemaphoreType.DMA,
        pltpu.SemaphoreType.DMA,
    ],
)()


_QC = 512

def _sc_headg_body(ql_hbm, qn_hbm, u_hbm, s_hbm, d_hbm,
                   u0_out, u1_out, sq_out, dq_out,
                   l_v, n_v, f0_v, f1_v, fs_v, r0_v, r1_v, sv_v, dv_v, sem):
    c = lax.axis_index("c")
    s = lax.axis_index("s")
    w = c * 16 + s

    def chunk(ch, _):
        base = w * (2 * _QC) + ch * _QC
        pltpu.sync_copy(ql_hbm.at[pl.ds(base, _QC)], l_v)
        pltpu.sync_copy(qn_hbm.at[pl.ds(base, _QC)], n_v)
        def fbody(i, _):
            l16 = l_v[pl.ds(i * 16, 16)]
            n16 = n_v[pl.ds(i * 16, 16)]
            fs_v[pl.ds(i * 16, 16)] = l16 * NN + n16
            f0 = l16 * (2 * NN) + n16
            f0_v[pl.ds(i * 16, 16)] = f0
            f1_v[pl.ds(i * 16, 16)] = f0 + NN
            return 0
        lax.fori_loop(0, _QC // 16, fbody, 0)
        pltpu.async_copy(u_hbm.at[f0_v], r0_v, sem).wait()
        pltpu.async_copy(u_hbm.at[f1_v], r1_v, sem).wait()
        pltpu.async_copy(s_hbm.at[fs_v], sv_v, sem).wait()
        pltpu.async_copy(d_hbm.at[fs_v], dv_v, sem).wait()
        pltpu.sync_copy(r0_v, u0_out.at[pl.ds(base, _QC)])
        pltpu.sync_copy(r1_v, u1_out.at[pl.ds(base, _QC)])
        pltpu.sync_copy(sv_v, sq_out.at[pl.ds(base, _QC)])
        pltpu.sync_copy(dv_v, dq_out.at[pl.ds(base, _QC)])
        return 0

    lax.fori_loop(0, 2, chunk, 0)


_sc_headg = functools.partial(
    pl.kernel, _sc_headg_body,
    compiler_params=_sc_params,
    out_type=[
        jax.ShapeDtypeStruct((2 * BB, 32), jnp.float32),
        jax.ShapeDtypeStruct((2 * BB, 32), jnp.float32),
        jax.ShapeDtypeStruct((2 * BB,), jnp.float32),
        jax.ShapeDtypeStruct((2 * BB,), jnp.float32),
    ],
    mesh=_mesh,
    scratch_types=[
        pltpu.VMEM((_QC,), jnp.int32),
        pltpu.VMEM((_QC,), jnp.int32),
        pltpu.VMEM((_QC,), jnp.int32),
        pltpu.VMEM((_QC,), jnp.int32),
        pltpu.VMEM((_QC,), jnp.int32),
        pltpu.VMEM((_QC, 32), jnp.float32),
        pltpu.VMEM((_QC, 32), jnp.float32),
        pltpu.VMEM((_QC,), jnp.float32),
        pltpu.VMEM((_QC,), jnp.float32),
        pltpu.SemaphoreType.DMA,
    ],
)()


def _tc_stats_body(t_ref, x_ref, dinv_ref, s_ref, st_ref):
    dinv = dinv_ref[...]
    sarr = dinv * t_ref[...] + dinv * dinv * x_ref[...]
    s_ref[...] = sarr
    st_ref[...] = jnp.stack(
        [jnp.sum(sarr, axis=1), jnp.sum(sarr * sarr, axis=1)], axis=1)


def _tc_stats(t, x, dinv):
    return pl.pallas_call(
        _tc_stats_body,
        out_shape=[
            jax.ShapeDtypeStruct((LL, NN), jnp.float32),
            jax.ShapeDtypeStruct((LL, 2), jnp.float32),
        ],
    )(t, x, dinv)


_GB = 2000

def _tc_g_body(s_ref, dinv_ref, a_ref, c_ref, g_ref):
    q = 2 * pl.program_id(0) + pl.program_id(1)
    ab = a_ref[q, :]
    cb = c_ref[q, :]
    sb = s_ref[...]
    db = dinv_ref[...]
    g_ref[...] = db * jnp.maximum(sb * ab[None, :] + cb[None, :], 0.0)


def _tc_g(s2d, dinv2d, a, c):
    return pl.pallas_call(
        _tc_g_body,
        grid=(LL, 2, NN // _GB),
        in_specs=[
            pl.BlockSpec((_GB, 1), lambda l, h, nb: (l * (NN // _GB) + nb, 0)),
            pl.BlockSpec((_GB, 1), lambda l, h, nb: (l * (NN // _GB) + nb, 0)),
            pl.BlockSpec((2 * LL, 32), lambda l, h, nb: (0, 0)),
            pl.BlockSpec((2 * LL, 32), lambda l, h, nb: (0, 0)),
        ],
        out_specs=pl.BlockSpec(
            (_GB, 32), lambda l, h, nb: ((2 * l + h) * (NN // _GB) + nb, 0)),
        out_shape=jax.ShapeDtypeStruct((2 * LL * NN, 32), jnp.float32),
    )(s2d, dinv2d, a, c)


_HB = 1024

def _tc_head_body(u0L, u0R, u1L, u1R, sqL, sqR, dqL, dqR,
                  lay_ref, ln_ref, rn_ref,
                  a_ref, c_ref, W2_ref, b2_ref,
                  gW1_ref, gb1_ref, gW2_ref, gb2_ref,
                  wsm_ref, wss_ref, wsb_ref,
                  dW_ref, db_ref, pW_ref, pb_ref,
                  p_ref, d_ref):
    lay = lay_ref[...]
    oh = (lay == jnp.arange(LL, dtype=jnp.int32)[None, :]).astype(jnp.float32)
    aR = oh @ a_ref[...]
    cR = oh @ c_ref[...]
    b2R = oh @ b2_ref[...]

    def side(u0, u1, sq, dq):
        u = jnp.concatenate([u0[...], u1[...]], axis=1)
        sv = sq[...]
        dv = dq[...]
        h = jnp.maximum(aR * sv + cR, 0.0)
        pre = dv * u + dv * dv * h
        emb = b2R
        for l in range(LL):
            emb = emb + oh[:, l:l + 1] * (pre @ W2_ref[l])
        return emb

    embL = side(u0L, u1L, sqL, dqL)
    embR = side(u0R, u1R, sqR, dqR)
    specific = jnp.concatenate([embL, embR], axis=1)
    common = (jnp.maximum(specific @ gW1_ref[...] + gb1_ref[...], 0.0)
              @ gW2_ref[...] + gb2_ref[...])
    layf = lay.astype(jnp.float32)
    lnf = ln_ref[...].astype(jnp.float32)
    rnf = rn_ref[...].astype(jnp.float32)
    wsm = wsm_ref[...]
    z = (layf * wsm[0:1, :] + lnf * wsm[1:2, :] + rnf * wsm[2:3, :]
         + specific @ wss_ref[...] + wsb_ref[...])
    z = z - jnp.max(z, axis=1, keepdims=True)
    ez = jnp.exp(z)
    wsf = ez / jnp.sum(ez, axis=1, keepdims=True)
    p_in = specific * wsf[:, 0:1] + common * wsf[:, 1:2]
    p_ref[...] = p_in @ pW_ref[...] + pb_ref[...]
    d_ref[...] = common @ dW_ref[...] + db_ref[...]


def _tc_head(u0, u1, sq, dq, lay2, ln2, rn2, a, c, W2, b2,
             gW1, gb1, gW2, gb2, wsm, wss, wsb, dW, db, pW, pb):
    nb = BB // _HB
    left = lambda i: (i, 0)
    right = lambda i: (i + nb, 0)
    full = lambda i: tuple([0] * 2)
    full3 = lambda i: (0, 0, 0)
    return pl.pallas_call(
        _tc_head_body,
        grid=(nb,),
        in_specs=[
            pl.BlockSpec((_HB, 32), left), pl.BlockSpec((_HB, 32), right),
            pl.BlockSpec((_HB, 32), left), pl.BlockSpec((_HB, 32), right),
            pl.BlockSpec((_HB, 1), left), pl.BlockSpec((_HB, 1), right),
            pl.BlockSpec((_HB, 1), left), pl.BlockSpec((_HB, 1), right),
            pl.BlockSpec((_HB, 1), left),
            pl.BlockSpec((_HB, 1), left),
            pl.BlockSpec((_HB, 1), left),
            pl.BlockSpec((LL, HH), full), pl.BlockSpec((LL, HH), full),
            pl.BlockSpec((LL, HH, HH), full3), pl.BlockSpec((LL, HH), full),
            pl.BlockSpec((2 * HH, 16), full), pl.BlockSpec((1, 16), full),
            pl.BlockSpec((16, 2 * HH), full), pl.BlockSpec((1, 2 * HH), full),
            pl.BlockSpec((3, 2), full), pl.BlockSpec((2 * HH, 2), full),
            pl.BlockSpec((1, 2), full),
            pl.BlockSpec((2 * HH, LL), full), pl.BlockSpec((1, LL), full),
            pl.BlockSpec((2 * HH, 2), full), pl.BlockSpec((1, 2), full),
        ],
        out_specs=[
            pl.BlockSpec((_HB, 2), left),
            pl.BlockSpec((_HB, LL), left),
        ],
        out_shape=[
            jax.ShapeDtypeStruct((BB, 2), jnp.float32),
            jax.ShapeDtypeStruct((BB, LL), jnp.float32),
        ],
    )(u0, u0, u1, u1, sq, sq, dq, dq, lay2[:BB], ln2, rn2, a, c, W2, b2,
      gW1, gb1, gW2, gb2, wsm, wss, wsb, dW, db, pW, pb)


def kernel(xs, edge_index, leftnode, rightnode, layer,
           W1, b1, gamma, beta, W2, b2,
           gW1, gb1, gW2, gb2, wsW, wsb, dW, db, pW, pb):
    x = xs[..., 0]
    src = edge_index[:, 0, :].reshape(LL * EE)
    dst = edge_index[:, 1, :].reshape(LL * EE)

    degp = _sc_deg(dst).reshape(2, LL, NN)
    deg = degp[0] + degp[1] + 1.0
    dinv = lax.rsqrt(deg)
    xd = (x * dinv).reshape(LL * NN)

    tp = _sc_txsum(src, dst, xd).reshape(2, LL, NN)
    t = tp[0] + tp[1]

    s_arr, st = _tc_stats(t, x, dinv)
    mean = st[:, 0] / NN
    var = st[:, 1] / NN - mean * mean
    w1r = W1[:, 0, :]
    a = gamma * w1r * lax.rsqrt(var[:, None] * w1r * w1r + 1e-5)
    c = beta - a * mean[:, None]

    g = _tc_g(s_arr.reshape(LL * NN, 1), dinv.reshape(LL * NN, 1),
              a.reshape(2 * LL, 32), c.reshape(2 * LL, 32))
    u = _sc_agg(src, dst, g)

    ql = jnp.concatenate([layer, layer])
    qn = jnp.concatenate([leftnode, rightnode])
    u0, u1, sq, dq = _sc_headg(
        ql, qn, u, s_arr.reshape(LL * NN), dinv.reshape(LL * NN))

    p_out, d_out = _tc_head(
        u0, u1, sq.reshape(2 * BB, 1), dq.reshape(2 * BB, 1),
        ql.reshape(2 * BB, 1), leftnode.reshape(BB, 1),
        rightnode.reshape(BB, 1),
        a, c, W2, b2,
        gW1, gb1.reshape(1, 16), gW2, gb2.reshape(1, 2 * HH),
        wsW[:3], wsW[3:], wsb.reshape(1, 2),
        dW, db.reshape(1, LL), pW, pb.reshape(1, 2))
    return (p_out, d_out)
